# baseline (speedup 1.0000x reference)
import functools

import jax
import jax.numpy as jnp
from jax import lax
from jax.experimental import pallas as pl
from jax.experimental.pallas import tpu as pltpu
from jax.experimental.pallas import tpu_sc as plsc

N_NODES = 10000
D = 128
N_LEAF = 8000
N_SPECIAL = 3
B, L = 4096, 200

NP = 10240
NE_PAD = 163840
E_PER_W = NE_PAD // 32
EC = 128
E_ROWS = E_PER_W // EC
BL = B * L
IDS_PER_W = BL // 32
CH = 128
NCH = IDS_PER_W // CH
TBL_ROWS = 9 * 1280


@functools.lru_cache(maxsize=None)
def _sc_mesh():
    return plsc.VectorSubcoreMesh(core_axis_name="c", subcore_axis_name="s")


def _wid(c, s):
    return c * 16 + s


@functools.lru_cache(maxsize=None)
def _deg_kernel():
    return pl.kernel(
        _deg_body, mesh=_sc_mesh(),
        compiler_params=pltpu.CompilerParams(needs_layout_passes=False),
        out_type=jax.ShapeDtypeStruct((2, NP), jnp.float32),
        scratch_types=[
            pltpu.VMEM_SHARED((NP,), jnp.float32),
            pltpu.VMEM((640,), jnp.float32),
            pltpu.VMEM((EC,), jnp.float32),
            pltpu.VMEM((E_ROWS, EC), jnp.int32),
        ],
    )


def _deg_body(dst2d_hbm, deg2_hbm, degacc, zbuf, ones, didx2):
    c = lax.axis_index("c")
    s = lax.axis_index("s")
    zero16 = jnp.zeros((16,), jnp.float32)
    for g in range(40):
        zbuf[pl.ds(g * 16, 16)] = zero16
    for g in range(EC // 16):
        ones[pl.ds(g * 16, 16)] = zero16 + 1.0
    pltpu.sync_copy(zbuf, degacc.at[pl.ds(s * 640, 640)])
    pltpu.sync_copy(dst2d_hbm.at[pl.ds(_wid(c, s) * E_ROWS, E_ROWS)], didx2)
    plsc.subcore_barrier()

    def body(j, carry):
        pltpu.sync_copy(ones, degacc.at[didx2.at[j]], add=True)
        return carry

    lax.fori_loop(0, E_ROWS, body, 0)
    plsc.subcore_barrier()
    pltpu.sync_copy(degacc.at[pl.ds(s * 640, 640)], deg2_hbm.at[c, pl.ds(s * 640, 640)])


def _tc1_body(x_ref, w_ref, degt_ref, out_ref):
    h = jnp.dot(x_ref[...], w_ref[...], preferred_element_type=jnp.float32)
    deg = degt_ref[:, 0:1] + degt_ref[:, 1:2] + 1.0
    dinv = lax.rsqrt(deg)
    out_ref[...] = h * dinv


def _tc1(x_pad, w, degt):
    return pl.pallas_call(
        _tc1_body,
        grid=(8,),
        in_specs=[
            pl.BlockSpec((1280, D), lambda i: (i, 0)),
            pl.BlockSpec((D, D), lambda i: (0, 0)),
            pl.BlockSpec((1280, 2), lambda i: (i, 0)),
        ],
        out_specs=pl.BlockSpec((1280, D), lambda i: (i, 0)),
        out_shape=jax.ShapeDtypeStruct((NP, D), jnp.float32),
    )(x_pad, w, degt)


@functools.lru_cache(maxsize=None)
def _msg_kernel():
    return pl.kernel(
        _msg_body, mesh=_sc_mesh(),
        compiler_params=pltpu.CompilerParams(needs_layout_passes=False),
        out_type=jax.ShapeDtypeStruct((2, NP, D), jnp.float32),
        scratch_types=[
            pltpu.VMEM_SHARED((NP, D), jnp.float32),
            pltpu.VMEM((E_ROWS, EC), jnp.int32),
            pltpu.VMEM((E_ROWS, EC), jnp.int32),
            [pltpu.VMEM((EC, D), jnp.float32)] * 2,
            [pltpu.SemaphoreType.DMA] * 2,
            [pltpu.SemaphoreType.DMA] * 2,
        ],
    )


def _msg_body(scaled_hbm, src2d_hbm, dst2d_hbm, acc2_hbm,
              acc, sidx2, didx2, rbs, gsems, ssems):
    c = lax.axis_index("c")
    s = lax.axis_index("s")
    w = (1 - c) * 16 + s
    pltpu.sync_copy(scaled_hbm.at[pl.ds(s * 640, 640)], acc.at[pl.ds(s * 640, 640)])
    pltpu.sync_copy(src2d_hbm.at[pl.ds(w * E_ROWS, E_ROWS)], sidx2)
    pltpu.sync_copy(dst2d_hbm.at[pl.ds(w * E_ROWS, E_ROWS)], didx2)
    plsc.subcore_barrier()

    def body(k, carry):
        base_row = 2 * k
        hg = [pltpu.async_copy(scaled_hbm.at[sidx2.at[base_row + t]], rbs[t],
                               gsems[t]) for t in range(2)]
        hs = []
        for t in range(2):
            hg[t].wait()
            hs.append(pltpu.async_copy(rbs[t], acc.at[didx2.at[base_row + t]],
                                       ssems[t], add=True))
        for h in hs:
            h.wait()
        return carry

    lax.fori_loop(0, E_ROWS // 2, body, 0)
    plsc.subcore_barrier()
    pltpu.sync_copy(acc.at[pl.ds(s * 640, 640)], acc2_hbm.at[c, pl.ds(s * 640, 640)])


def _tc2_body(acc2_ref, scaled_ref, degt_ref, sp_ref, b_ref, out_ref):
    pid = pl.program_id(0)

    @pl.when(pid == 8)
    def _():
        out_ref[...] = sp_ref[...]

    @pl.when(pid < 8)
    def _():
        a = acc2_ref[0] + acc2_ref[1] - scaled_ref[...]
        deg = degt_ref[:, 0:1] + degt_ref[:, 1:2] + 1.0
        dinv = lax.rsqrt(deg)
        out_ref[...] = a * dinv + b_ref[...]


def _tc2(acc2, scaled, degt, sp_pad, b2):
    return pl.pallas_call(
        _tc2_body,
        grid=(9,),
        in_specs=[
            pl.BlockSpec((2, 1280, D), lambda i: (0, jnp.minimum(i, 7), 0)),
            pl.BlockSpec((1280, D), lambda i: (jnp.minimum(i, 7), 0)),
            pl.BlockSpec((1280, 2), lambda i: (jnp.minimum(i, 7), 0)),
            pl.BlockSpec((1280, D), lambda i: (0, 0)),
            pl.BlockSpec((1, D), lambda i: (0, 0)),
        ],
        out_specs=pl.BlockSpec((1280, D), lambda i: (i, 0)),
        out_shape=jax.ShapeDtypeStruct((TBL_ROWS, D), jnp.float32),
    )(acc2, scaled, degt, sp_pad, b2)


@functools.lru_cache(maxsize=None)
def _lookup_kernel():
    return pl.kernel(
        _lookup_body, mesh=_sc_mesh(),
        compiler_params=pltpu.CompilerParams(needs_layout_passes=False),
        out_type=jax.ShapeDtypeStruct((BL, D), jnp.float32),
        scratch_types=[
            pltpu.VMEM((N_LEAF,), jnp.int32),
            pltpu.VMEM((NCH + 4, EC), jnp.int32),
            pltpu.VMEM((4 * CH, D), jnp.float32),
            [pltpu.SemaphoreType.DMA] * 4,
            pltpu.SemaphoreType.DMA,
        ],
    )


def _lookup_body(ctable_hbm, ids2d_hbm, leaf_hbm, out_hbm,
                 leafbuf, idsbuf, buf, gsems, osem):
    c = lax.axis_index("c")
    s = lax.axis_index("s")
    w = _wid(c, s)
    base = w * IDS_PER_W
    pltpu.sync_copy(leaf_hbm, leafbuf)
    pltpu.sync_copy(ids2d_hbm.at[pl.ds(w * NCH, NCH)], idsbuf.at[pl.ds(0, NCH)])

    def remap_chunk(ch):
        rd = jnp.minimum(ch, NCH - 1)

        def grp(g, carry):
            ids16 = idsbuf[rd, pl.ds(g * 16, 16)]
            li = jnp.maximum(ids16 - N_SPECIAL, 0)
            lv = plsc.load_gather(leafbuf, [li])
            row = jnp.where(ids16 < N_SPECIAL, ids16 + NP, lv)
            idsbuf[ch, pl.ds(g * 16, 16)] = row
            return carry

        lax.fori_loop(0, 8, grp, 0)

    for t in range(4):
        remap_chunk(t)

    def body(k, carry):
        cb = 4 * k
        hg = [pltpu.async_copy(ctable_hbm.at[idsbuf.at[cb + t]],
                               buf.at[pl.ds(t * CH, CH)], gsems[t])
              for t in range(4)]
        for t in range(4):
            remap_chunk(cb + 4 + t)
        for t in range(4):
            hg[t].wait()
        pltpu.async_copy(buf, out_hbm.at[pl.ds(base + cb * CH, 4 * CH)],
                         osem).wait()
        return carry

    lax.fori_loop(0, NCH // 4, body, 0)


def kernel(input_ids, special_embedding, node_emb, edge_index, leaf_idx, W, b):
    src = edge_index[0].astype(jnp.int32)
    dst = edge_index[1].astype(jnp.int32)
    pad = NE_PAD - src.shape[0]
    src2d = jnp.pad(src, (0, pad)).reshape(NE_PAD // EC, EC)
    pad_dst = N_NODES + jnp.arange(pad, dtype=jnp.int32) % (NP - N_NODES)
    dst2d = jnp.concatenate([dst, pad_dst]).reshape(NE_PAD // EC, EC)
    x_pad = jnp.pad(node_emb, ((0, NP - N_NODES), (0, 0)))
    ids_flat = input_ids.reshape(-1).astype(jnp.int32)
    leaf32 = leaf_idx.astype(jnp.int32)

    deg2 = _deg_kernel()(dst2d)
    degt = jnp.transpose(deg2)
    scaled = _tc1(x_pad, W, degt)
    acc2 = _msg_kernel()(scaled, src2d, dst2d)
    sp_pad = jnp.pad(special_embedding, ((0, 1280 - N_SPECIAL), (0, 0)))
    ctable = _tc2(acc2, scaled, degt, sp_pad, b.reshape(1, D))
    ids2d = ids_flat.reshape(BL // EC, EC)
    out = _lookup_kernel()(ctable, ids2d, leaf32)
    return out.reshape(B, L, D)


# speedup vs baseline: 1.1019x; 1.1019x over previous
import functools

import jax
import jax.numpy as jnp
from jax import lax
from jax.experimental import pallas as pl
from jax.experimental.pallas import tpu as pltpu
from jax.experimental.pallas import tpu_sc as plsc

N_NODES = 10000
D = 128
N_LEAF = 8000
N_SPECIAL = 3
B, L = 4096, 200

NP = 10240
NE_PAD = 163840
E_PER_W = NE_PAD // 32
EC = 128
E_ROWS = E_PER_W // EC
BL = B * L
IDS_PER_W = BL // 32
CH = 128
NCH = IDS_PER_W // CH
TBL_ROWS = 9 * 1280


@functools.lru_cache(maxsize=None)
def _sc_mesh():
    return plsc.VectorSubcoreMesh(core_axis_name="c", subcore_axis_name="s")


def _wid(c, s):
    return c * 16 + s


@functools.lru_cache(maxsize=None)
def _deg_kernel():
    return pl.kernel(
        _deg_body, mesh=_sc_mesh(),
        compiler_params=pltpu.CompilerParams(needs_layout_passes=False),
        out_type=jax.ShapeDtypeStruct((2, NP), jnp.float32),
        scratch_types=[
            pltpu.VMEM_SHARED((NP,), jnp.float32),
            pltpu.VMEM((640,), jnp.float32),
            pltpu.VMEM((EC,), jnp.float32),
            pltpu.VMEM((E_ROWS, EC), jnp.int32),
        ],
    )


def _deg_body(dst2d_hbm, deg2_hbm, degacc, zbuf, ones, didx2):
    c = lax.axis_index("c")
    s = lax.axis_index("s")
    zero16 = jnp.zeros((16,), jnp.float32)
    for g in range(40):
        zbuf[pl.ds(g * 16, 16)] = zero16
    for g in range(EC // 16):
        ones[pl.ds(g * 16, 16)] = zero16 + 1.0
    pltpu.sync_copy(zbuf, degacc.at[pl.ds(s * 640, 640)])
    pltpu.sync_copy(dst2d_hbm.at[pl.ds(_wid(c, s) * E_ROWS, E_ROWS)], didx2)
    plsc.subcore_barrier()

    def body(j, carry):
        pltpu.sync_copy(ones, degacc.at[didx2.at[j]], add=True)
        return carry

    lax.fori_loop(0, E_ROWS, body, 0)
    plsc.subcore_barrier()
    pltpu.sync_copy(degacc.at[pl.ds(s * 640, 640)], deg2_hbm.at[c, pl.ds(s * 640, 640)])


def _tc1_body(x_ref, w_ref, degt_ref, out_ref):
    h = jnp.dot(x_ref[...], w_ref[...], preferred_element_type=jnp.float32)
    deg = degt_ref[:, 0:1] + degt_ref[:, 1:2] + 1.0
    dinv = lax.rsqrt(deg)
    out_ref[...] = h * dinv


def _tc1(x_pad, w, degt):
    return pl.pallas_call(
        _tc1_body,
        grid=(8,),
        in_specs=[
            pl.BlockSpec((1280, D), lambda i: (i, 0)),
            pl.BlockSpec((D, D), lambda i: (0, 0)),
            pl.BlockSpec((1280, 2), lambda i: (i, 0)),
        ],
        out_specs=pl.BlockSpec((1280, D), lambda i: (i, 0)),
        out_shape=jax.ShapeDtypeStruct((NP, D), jnp.float32),
    )(x_pad, w, degt)


@functools.lru_cache(maxsize=None)
def _msg_kernel():
    return pl.kernel(
        _msg_body, mesh=_sc_mesh(),
        compiler_params=pltpu.CompilerParams(needs_layout_passes=False),
        out_type=jax.ShapeDtypeStruct((2, NP, D), jnp.float32),
        scratch_types=[
            pltpu.VMEM_SHARED((NP, D), jnp.float32),
            pltpu.VMEM((64, EC), jnp.int32),
            pltpu.VMEM((64, EC), jnp.int32),
            [pltpu.VMEM((EC, D), jnp.float32)] * 2,
            [pltpu.SemaphoreType.DMA] * 2,
            [pltpu.SemaphoreType.DMA] * 2,
        ],
    )


R_C0 = 64
R_C1 = 80 - R_C0


def _msg_body(scaled_hbm, src2d_hbm, dst2d_hbm, acc2_hbm,
              acc, sidx2, didx2, rbs, gsems, ssems):
    c = lax.axis_index("c")
    s = lax.axis_index("s")
    pltpu.sync_copy(scaled_hbm.at[pl.ds(s * 640, 640)], acc.at[pl.ds(s * 640, 640)])
    nrows = jnp.where(c == 0, R_C0, R_C1)
    row0 = jnp.where(c == 0, s * R_C0, 16 * R_C0 + s * R_C1)
    pltpu.sync_copy(src2d_hbm.at[pl.ds(row0, 64)], sidx2.at[pl.ds(0, 64)])
    pltpu.sync_copy(dst2d_hbm.at[pl.ds(row0, 64)], didx2.at[pl.ds(0, 64)])
    plsc.subcore_barrier()

    def body(k, carry):
        base_row = 2 * k
        hg = [pltpu.async_copy(scaled_hbm.at[sidx2.at[base_row + t]], rbs[t],
                               gsems[t]) for t in range(2)]
        hs = []
        for t in range(2):
            hg[t].wait()
            hs.append(pltpu.async_copy(rbs[t], acc.at[didx2.at[base_row + t]],
                                       ssems[t], add=True))
        for h in hs:
            h.wait()
        return carry

    lax.fori_loop(0, nrows // 2, body, 0)
    plsc.subcore_barrier()
    pltpu.sync_copy(acc.at[pl.ds(s * 640, 640)], acc2_hbm.at[c, pl.ds(s * 640, 640)])


def _tc2_body(acc2_ref, scaled_ref, degt_ref, sp_ref, b_ref, out_ref):
    pid = pl.program_id(0)

    @pl.when(pid == 8)
    def _():
        out_ref[...] = sp_ref[...]

    @pl.when(pid < 8)
    def _():
        a = acc2_ref[0] + acc2_ref[1] - scaled_ref[...]
        deg = degt_ref[:, 0:1] + degt_ref[:, 1:2] + 1.0
        dinv = lax.rsqrt(deg)
        out_ref[...] = a * dinv + b_ref[...]


def _tc2(acc2, scaled, degt, sp_pad, b2):
    return pl.pallas_call(
        _tc2_body,
        grid=(9,),
        in_specs=[
            pl.BlockSpec((2, 1280, D), lambda i: (0, jnp.minimum(i, 7), 0)),
            pl.BlockSpec((1280, D), lambda i: (jnp.minimum(i, 7), 0)),
            pl.BlockSpec((1280, 2), lambda i: (jnp.minimum(i, 7), 0)),
            pl.BlockSpec((1280, D), lambda i: (0, 0)),
            pl.BlockSpec((1, D), lambda i: (0, 0)),
        ],
        out_specs=pl.BlockSpec((1280, D), lambda i: (i, 0)),
        out_shape=jax.ShapeDtypeStruct((TBL_ROWS, D), jnp.float32),
    )(acc2, scaled, degt, sp_pad, b2)


@functools.lru_cache(maxsize=None)
def _lookup_kernel():
    return pl.kernel(
        _lookup_body, mesh=_sc_mesh(),
        compiler_params=pltpu.CompilerParams(needs_layout_passes=False),
        out_type=jax.ShapeDtypeStruct((BL, D), jnp.float32),
        scratch_types=[
            pltpu.VMEM((N_LEAF,), jnp.int32),
            pltpu.VMEM((NCH + 4, EC), jnp.int32),
            pltpu.VMEM((4 * CH, D), jnp.float32),
            [pltpu.SemaphoreType.DMA] * 4,
            pltpu.SemaphoreType.DMA,
        ],
    )


def _lookup_body(ctable_hbm, ids2d_hbm, leaf_hbm, out_hbm,
                 leafbuf, idsbuf, buf, gsems, osem):
    c = lax.axis_index("c")
    s = lax.axis_index("s")
    w = _wid(c, s)
    base = w * IDS_PER_W
    pltpu.sync_copy(leaf_hbm, leafbuf)
    pltpu.sync_copy(ids2d_hbm.at[pl.ds(w * NCH, NCH)], idsbuf.at[pl.ds(0, NCH)])

    def remap_chunk(ch):
        rd = jnp.minimum(ch, NCH - 1)

        def grp(g, carry):
            ids16 = idsbuf[rd, pl.ds(g * 16, 16)]
            li = jnp.maximum(ids16 - N_SPECIAL, 0)
            lv = plsc.load_gather(leafbuf, [li])
            row = jnp.where(ids16 < N_SPECIAL, ids16 + NP, lv)
            idsbuf[ch, pl.ds(g * 16, 16)] = row
            return carry

        lax.fori_loop(0, 8, grp, 0)

    for t in range(4):
        remap_chunk(t)

    def body(k, carry):
        cb = 4 * k
        hg = [pltpu.async_copy(ctable_hbm.at[idsbuf.at[cb + t]],
                               buf.at[pl.ds(t * CH, CH)], gsems[t])
              for t in range(4)]
        for t in range(4):
            remap_chunk(cb + 4 + t)
        for t in range(4):
            hg[t].wait()
        pltpu.async_copy(buf, out_hbm.at[pl.ds(base + cb * CH, 4 * CH)],
                         osem).wait()
        return carry

    lax.fori_loop(0, NCH // 4, body, 0)


def kernel(input_ids, special_embedding, node_emb, edge_index, leaf_idx, W, b):
    src = edge_index[0].astype(jnp.int32)
    dst = edge_index[1].astype(jnp.int32)
    full = (16 * R_C0 + 16 * R_C1 + 64) * EC
    pad = full - src.shape[0]
    src2d = jnp.pad(src, (0, pad)).reshape(full // EC, EC)
    pad_dst = N_NODES + jnp.arange(pad, dtype=jnp.int32) % (NP - N_NODES)
    dst2d = jnp.concatenate([dst, pad_dst]).reshape(full // EC, EC)
    x_pad = jnp.pad(node_emb, ((0, NP - N_NODES), (0, 0)))
    ids_flat = input_ids.reshape(-1).astype(jnp.int32)
    leaf32 = leaf_idx.astype(jnp.int32)

    deg2 = _deg_kernel()(dst2d)
    degt = jnp.transpose(deg2)
    scaled = _tc1(x_pad, W, degt)
    acc2 = _msg_kernel()(scaled, src2d, dst2d)
    sp_pad = jnp.pad(special_embedding, ((0, 1280 - N_SPECIAL), (0, 0)))
    ctable = _tc2(acc2, scaled, degt, sp_pad, b.reshape(1, D))
    ids2d = ids_flat.reshape(BL // EC, EC)
    out = _lookup_kernel()(ctable, ids2d, leaf32)
    return out.reshape(B, L, D)
